# Initial kernel scaffold; baseline (speedup 1.0000x reference)
import functools

import jax
import jax.numpy as jnp
from jax import lax
from jax.experimental import pallas as pl
from jax.experimental.pallas import tpu as pltpu
from jax.experimental.pallas import tpu_sc as plsc

N = 10000
E = 320000
DIM = 128
NB = 128
L = 3

NC = 2
NS = 16
NW = NC * NS
EPW = E // NW
CH = 80
NCHUNK = EPW // CH
RPT = N // NS
ZR = 125

_sc_mesh = plsc.VectorSubcoreMesh(core_axis_name="c", subcore_axis_name="s")


def _sc_agg_body(table_hbm, src_hbm, dst_hbm, out_hbm, idx_s, idx_d, rows,
                 zbuf, acc):
    c = lax.axis_index("c")
    s = lax.axis_index("s")
    wid = s * NC + c

    @pl.loop(0, ZR)
    def _(r):
        @pl.loop(0, DIM, step=16)
        def _(cc):
            zbuf[r, pl.ds(cc, 16)] = jnp.zeros((16,), jnp.float32)

    @pl.loop(0, RPT // ZR)
    def _(k):
        pltpu.sync_copy(zbuf, acc.at[pl.ds(s * RPT + k * ZR, ZR)])

    pltpu.sync_copy(src_hbm.at[wid], idx_s)
    pltpu.sync_copy(dst_hbm.at[wid], idx_d)
    plsc.subcore_barrier()

    @pl.loop(0, NCHUNK)
    def _(j):
        pltpu.sync_copy(table_hbm.at[idx_s.at[j]], rows)
        pltpu.sync_copy(rows, acc.at[idx_d.at[j]], add=True)

    plsc.subcore_barrier()

    @pl.loop(0, RPT // ZR)
    def _(k):
        b = s * RPT + k * ZR
        pltpu.sync_copy(acc.at[pl.ds(b, ZR)], out_hbm.at[c].at[pl.ds(b, ZR)])


_sc_agg = pl.kernel(
    _sc_agg_body,
    out_type=jax.ShapeDtypeStruct((NC, N, DIM), jnp.float32),
    mesh=_sc_mesh,
    scratch_types=[
        pltpu.VMEM((NCHUNK, CH), jnp.int32),
        pltpu.VMEM((NCHUNK, CH), jnp.int32),
        pltpu.VMEM((CH, DIM), jnp.float32),
        pltpu.VMEM((ZR, DIM), jnp.float32),
        pltpu.VMEM_SHARED((N, DIM), jnp.float32),
    ],
)


def _sc_edge_body(s_hbm, b_hbm, src_hbm, dst_hbm, out_hbm, s_vm, b_vm, idx_s,
                  idx_d, rowbuf, zbuf, acc):
    c = lax.axis_index("c")
    s = lax.axis_index("s")
    wid = s * NC + c

    @pl.loop(0, RPT)
    def _(r):
        zbuf[r, :] = jnp.zeros((16,), jnp.float32)

    @pl.loop(0, CH)
    def _(r):
        rowbuf[r, :] = jnp.zeros((16,), jnp.float32)

    pltpu.sync_copy(zbuf, acc.at[pl.ds(s * RPT, RPT)])
    pltpu.sync_copy(s_hbm, s_vm)
    pltpu.sync_copy(b_hbm, b_vm)
    pltpu.sync_copy(src_hbm.at[wid], idx_s)
    pltpu.sync_copy(dst_hbm.at[wid], idx_d)
    plsc.subcore_barrier()

    col0 = jnp.zeros((16,), jnp.int32)
    col1 = col0 + 1

    @pl.loop(0, NCHUNK)
    def _(j):
        @pl.loop(0, CH // 16)
        def _(g):
            srcv = idx_s[j, pl.ds(g * 16, 16)]
            dstv = idx_d[j, pl.ds(g * 16, 16)]
            sv = plsc.load_gather(s_vm, [srcv])
            dv = plsc.load_gather(s_vm, [dstv])
            bv = plsc.load_gather(b_vm, [dstv])
            e = jnp.exp(sv * dv - bv)
            rowv = lax.iota(jnp.int32, (16,)) + g * 16
            plsc.store_scatter(rowbuf, [rowv, col0], e)
            plsc.store_scatter(rowbuf, [rowv, col1], sv * e)

        pltpu.sync_copy(rowbuf, acc.at[idx_d.at[j]], add=True)

    plsc.subcore_barrier()
    pltpu.sync_copy(acc.at[pl.ds(s * RPT, RPT)],
                    out_hbm.at[c].at[pl.ds(s * RPT, RPT)])


_sc_edge = pl.kernel(
    _sc_edge_body,
    out_type=jax.ShapeDtypeStruct((NC, N, 16), jnp.float32),
    mesh=_sc_mesh,
    scratch_types=[
        pltpu.VMEM((N,), jnp.float32),
        pltpu.VMEM((N,), jnp.float32),
        pltpu.VMEM((NCHUNK, CH), jnp.int32),
        pltpu.VMEM((NCHUNK, CH), jnp.int32),
        pltpu.VMEM((CH, 16), jnp.float32),
        pltpu.VMEM((RPT, 16), jnp.float32),
        pltpu.VMEM_SHARED((N, 16), jnp.float32),
    ],
)


def _dot(a, b):
    return lax.dot_general(a, b, (((1,), (0,)), ((), ())),
                           precision=lax.Precision.HIGHEST,
                           preferred_element_type=jnp.float32)


def _mlp_body(h_ref, p_ref, w1t_ref, b1_ref, w2t_ref, b2_ref, mw_ref,
              h_out, np_out, *, last):
    h2 = h_ref[...] + p_ref[0] + p_ref[1]
    a = jnp.maximum(_dot(h2, w1t_ref[...]) + b1_ref[...], 0.0)
    o = _dot(a, w2t_ref[...]) + b2_ref[...]
    if not last:
        o = jnp.maximum(o, 0.0)
    h_out[...] = o
    np_out[...] = _dot(o, mw_ref[...])


def _mlp(h, parts, w1t, b1, w2t, b2, mw, last):
    grid = 8
    blk = N // grid
    return pl.pallas_call(
        functools.partial(_mlp_body, last=last),
        grid=(grid,),
        in_specs=[
            pl.BlockSpec((blk, DIM), lambda i: (i, 0)),
            pl.BlockSpec((NC, blk, DIM), lambda i: (0, i, 0)),
            pl.BlockSpec((DIM, DIM), lambda i: (0, 0)),
            pl.BlockSpec((1, DIM), lambda i: (0, 0)),
            pl.BlockSpec((DIM, DIM), lambda i: (0, 0)),
            pl.BlockSpec((1, DIM), lambda i: (0, 0)),
            pl.BlockSpec((DIM, 1), lambda i: (0, 0)),
        ],
        out_specs=[
            pl.BlockSpec((blk, DIM), lambda i: (i, 0)),
            pl.BlockSpec((blk, 1), lambda i: (i, 0)),
        ],
        out_shape=[
            jax.ShapeDtypeStruct((N, DIM), jnp.float32),
            jax.ShapeDtypeStruct((N, 1), jnp.float32),
        ],
    )(h, parts, w1t, b1, w2t, b2, mw)


def _prep_body(np0_ref, np1_ref, np2_ref, b_ref, s_out, oh_out):
    s = np0_ref[...] + np1_ref[...] + np2_ref[...] + b_ref[0, 0]
    amax = jnp.max(s)
    amin = jnp.min(s)
    w = (amax - amin) / 128.0 + 1e-30
    bk = jnp.clip(((s - amin) / w).astype(jnp.int32), 0, 127)
    iot = lax.broadcasted_iota(jnp.int32, (1, 128), 1)
    s_out[...] = s
    oh_out[...] = (bk == iot).astype(jnp.float32)


def _prep(np0, np1, np2, mlp_b):
    return pl.pallas_call(
        _prep_body,
        out_shape=[
            jax.ShapeDtypeStruct((N, 1), jnp.float32),
            jax.ShapeDtypeStruct((N, DIM), jnp.float32),
        ],
    )(np0, np1, np2, mlp_b)


def _bound_body(s_ref, cnt_ref, b_out):
    s = s_ref[...]
    cnt = cnt_ref[0] + cnt_ref[1]
    amax = jnp.max(s)
    amin = jnp.min(s)
    w = (amax - amin) / 128.0 + 1e-30
    iot = lax.broadcasted_iota(jnp.float32, (1, 128), 1)
    nz = cnt > 0.0
    kmax = jnp.max(jnp.where(nz, iot, -1.0), axis=1, keepdims=True)
    kmin = jnp.min(jnp.where(nz, iot, 200.0), axis=1, keepdims=True)
    ub = amin + w * (kmax + 1.0)
    lb = amin + w * kmin
    b_out[...] = jnp.maximum(s * ub, s * lb)


def _bound(s, cparts):
    return pl.pallas_call(
        _bound_body,
        out_shape=jax.ShapeDtypeStruct((N, 1), jnp.float32),
    )(s, cparts)


def _final_body(s_ref, ep_ref, batch_ref, out):
    denom = ep_ref[0, :, 0:1] + ep_ref[1, :, 0:1]
    num = ep_ref[0, :, 1:2] + ep_ref[1, :, 1:2]
    new = jnp.where(denom > 0.0, num / denom, 0.0)
    v = s_ref[...] + new
    iot = lax.broadcasted_iota(jnp.int32, (1, NB), 1)
    oh = batch_ref[...] == iot
    neg = jnp.float32(-3.0e38)
    m_b = jnp.max(jnp.where(oh, v, neg), axis=0, keepdims=True)
    m_n = jnp.max(jnp.where(oh, m_b, neg), axis=1, keepdims=True)
    e = jnp.exp(v - m_n)
    d_b = jnp.sum(jnp.where(oh, e, 0.0), axis=0, keepdims=True)
    d_n = jnp.sum(jnp.where(oh, d_b, 0.0), axis=1, keepdims=True)
    out[...] = e / (d_n + 1e-16)


def _final(s, eparts, batch2d):
    return pl.pallas_call(
        _final_body,
        out_shape=jax.ShapeDtypeStruct((N, 1), jnp.float32),
    )(s, eparts, batch2d)


def kernel(x, edge_index, batch, W1_0, b1_0, W2_0, b2_0, W1_1, b1_1, W2_1,
           b2_1, W1_2, b1_2, W2_2, b2_2, mlp_W, mlp_b):
    src = edge_index[0].reshape(NW, NCHUNK, CH)
    dst = edge_index[1].reshape(NW, NCHUNK, CH)
    layers = [(W1_0, b1_0, W2_0, b2_0), (W1_1, b1_1, W2_1, b2_1),
              (W1_2, b1_2, W2_2, b2_2)]

    h = x
    nps = []
    for i, (W1, b1, W2, b2) in enumerate(layers):
        parts = _sc_agg(h, src, dst)
        mw = mlp_W[0, i * DIM:(i + 1) * DIM].reshape(DIM, 1)
        h, np_i = _mlp(h, parts, W1.T, b1.reshape(1, DIM), W2.T,
                       b2.reshape(1, DIM), mw, last=(i == L - 1))
        nps.append(np_i)

    s, onehot = _prep(nps[0], nps[1], nps[2], mlp_b.reshape(1, 1))
    cparts = _sc_agg(onehot, src, dst)
    bound = _bound(s, cparts)
    eparts = _sc_edge(s.reshape(N), bound.reshape(N), src, dst)
    return _final(s, eparts, batch.reshape(N, 1))


# speedup vs baseline: 14.6239x; 14.6239x over previous
import functools

import jax
import jax.numpy as jnp
from jax import lax
from jax.experimental import pallas as pl
from jax.experimental.pallas import tpu as pltpu
from jax.experimental.pallas import tpu_sc as plsc

N = 10000
E = 320000
DIM = 128
NB = 128
L = 3

NC = 2
NS = 16
NW = NC * NS
EPW = E // NW
CH = 80
NCHUNK = EPW // CH
CHA = 125
NCA = EPW // CHA
IB = 16
NPAD = 10240
RPT = NPAD // NS
ZR = 32

_sc_mesh = plsc.VectorSubcoreMesh(core_axis_name="c", subcore_axis_name="s")


def _sc_agg_body(table_hbm, src_hbm, dst_hbm, out_hbm, idx_s, idx_d, rows,
                 zbuf, acc):
    c = lax.axis_index("c")
    s = lax.axis_index("s")
    wid = s * NC + c

    @pl.loop(0, ZR)
    def _(r):
        @pl.loop(0, DIM, step=16)
        def _(cc):
            zbuf[r, pl.ds(cc, 16)] = jnp.zeros((16,), jnp.float32)

    @pl.loop(0, RPT // ZR)
    def _(k):
        pltpu.sync_copy(zbuf, acc.at[pl.ds(s * RPT + k * ZR, ZR)])

    plsc.subcore_barrier()

    @pl.loop(0, NCA // IB)
    def _(bb):
        pltpu.sync_copy(src_hbm.at[wid].at[pl.ds(bb * IB, IB)], idx_s)
        pltpu.sync_copy(dst_hbm.at[wid].at[pl.ds(bb * IB, IB)], idx_d)

        @pl.loop(0, IB)
        def _(j):
            pltpu.sync_copy(table_hbm.at[idx_s.at[j]], rows)
            pltpu.sync_copy(rows, acc.at[idx_d.at[j]], add=True)

    plsc.subcore_barrier()

    @pl.loop(0, RPT // ZR)
    def _(k):
        b = s * RPT + k * ZR
        pltpu.sync_copy(acc.at[pl.ds(b, ZR)], out_hbm.at[c].at[pl.ds(b, ZR)])


_sc_agg = pl.kernel(
    _sc_agg_body,
    out_type=pltpu.HBM((NC, NPAD, DIM), jnp.float32),
    mesh=_sc_mesh,
    scratch_types=[
        pltpu.VMEM((IB, CHA), jnp.int32),
        pltpu.VMEM((IB, CHA), jnp.int32),
        pltpu.VMEM((CHA, DIM), jnp.float32),
        pltpu.VMEM((ZR, DIM), jnp.float32),
        pltpu.VMEM_SHARED((NPAD, DIM), jnp.float32),
    ],
)


def _sc_edge_body(s_hbm, b_hbm, src_hbm, dst_hbm, out_hbm, s_vm, b_vm, idx_s,
                  idx_d, rowbuf, zbuf, acc):
    c = lax.axis_index("c")
    s = lax.axis_index("s")
    wid = s * NC + c

    @pl.loop(0, RPT)
    def _(r):
        zbuf[r, :] = jnp.zeros((16,), jnp.float32)

    @pl.loop(0, CH)
    def _(r):
        rowbuf[r, :] = jnp.zeros((16,), jnp.float32)

    pltpu.sync_copy(zbuf, acc.at[pl.ds(s * RPT, RPT)])
    pltpu.sync_copy(s_hbm, s_vm)
    pltpu.sync_copy(b_hbm, b_vm)
    pltpu.sync_copy(src_hbm.at[wid], idx_s)
    pltpu.sync_copy(dst_hbm.at[wid], idx_d)
    plsc.subcore_barrier()

    col0 = jnp.zeros((16,), jnp.int32)
    col1 = col0 + 1

    @pl.loop(0, NCHUNK)
    def _(j):
        @pl.loop(0, CH // 16)
        def _(g):
            srcv = idx_s[j, pl.ds(g * 16, 16)]
            dstv = idx_d[j, pl.ds(g * 16, 16)]
            sv = plsc.load_gather(s_vm, [srcv])
            dv = plsc.load_gather(s_vm, [dstv])
            bv = plsc.load_gather(b_vm, [dstv])
            e = jnp.exp(sv * dv - bv)
            rowv = lax.iota(jnp.int32, 16) + g * 16
            plsc.store_scatter(rowbuf, [rowv, col0], e)
            plsc.store_scatter(rowbuf, [rowv, col1], sv * e)

        pltpu.sync_copy(rowbuf, acc.at[idx_d.at[j]], add=True)

    plsc.subcore_barrier()
    pltpu.sync_copy(acc.at[pl.ds(s * RPT, RPT)],
                    out_hbm.at[c].at[pl.ds(s * RPT, RPT)])


_sc_edge = pl.kernel(
    _sc_edge_body,
    out_type=pltpu.HBM((NC, NPAD, 16), jnp.float32),
    mesh=_sc_mesh,
    scratch_types=[
        pltpu.VMEM((N,), jnp.float32),
        pltpu.VMEM((N,), jnp.float32),
        pltpu.VMEM((NCHUNK, CH), jnp.int32),
        pltpu.VMEM((NCHUNK, CH), jnp.int32),
        pltpu.VMEM((CH, 16), jnp.float32),
        pltpu.VMEM((RPT, 16), jnp.float32),
        pltpu.VMEM_SHARED((NPAD, 16), jnp.float32),
    ],
    compiler_params=pltpu.CompilerParams(needs_layout_passes=False, use_tc_tiling_on_sc=False),
)


def _dot(a, b):
    return lax.dot_general(a, b, (((1,), (0,)), ((), ())),
                           preferred_element_type=jnp.float32)


def _mlp_body(h_ref, p_ref, w1t_ref, b1_ref, w2t_ref, b2_ref, mw_ref,
              h_out, np_out, *, last):
    h2 = h_ref[...] + p_ref[0] + p_ref[1]
    a = jnp.maximum(_dot(h2, w1t_ref[...]) + b1_ref[...], 0.0)
    o = _dot(a, w2t_ref[...]) + b2_ref[...]
    if not last:
        o = jnp.maximum(o, 0.0)
    h_out[...] = o
    np_out[...] = _dot(o, mw_ref[...])


def _mlp(h, parts, w1t, b1, w2t, b2, mw, last):
    grid = 10
    blk = N // grid
    return pl.pallas_call(
        functools.partial(_mlp_body, last=last),
        grid=(grid,),
        in_specs=[
            pl.BlockSpec((blk, DIM), lambda i: (i, 0)),
            pl.BlockSpec((NC, blk, DIM), lambda i: (0, i, 0)),
            pl.BlockSpec((DIM, DIM), lambda i: (0, 0)),
            pl.BlockSpec((1, DIM), lambda i: (0, 0)),
            pl.BlockSpec((DIM, DIM), lambda i: (0, 0)),
            pl.BlockSpec((1, DIM), lambda i: (0, 0)),
            pl.BlockSpec((DIM, 1), lambda i: (0, 0)),
        ],
        out_specs=[
            pl.BlockSpec((blk, DIM), lambda i: (i, 0)),
            pl.BlockSpec((blk, 1), lambda i: (i, 0)),
        ],
        out_shape=[
            jax.ShapeDtypeStruct((N, DIM), jnp.float32),
            jax.ShapeDtypeStruct((N, 1), jnp.float32),
        ],
    )(h, parts, w1t, b1, w2t, b2, mw)


def _prep_body(np0_ref, np1_ref, np2_ref, b_ref, s_out, oh_out):
    s = np0_ref[...] + np1_ref[...] + np2_ref[...] + b_ref[0, 0]
    amax = jnp.max(s)
    amin = jnp.min(s)
    w = (amax - amin) / 128.0 + 1e-30
    bk = jnp.clip(((s - amin) / w).astype(jnp.int32), 0, 127)
    iot = lax.broadcasted_iota(jnp.int32, (1, 128), 1)
    s_out[...] = s
    oh_out[...] = (bk == iot).astype(jnp.float32)


def _prep(np0, np1, np2, mlp_b):
    return pl.pallas_call(
        _prep_body,
        out_shape=[
            jax.ShapeDtypeStruct((N, 1), jnp.float32),
            jax.ShapeDtypeStruct((N, DIM), jnp.float32),
        ],
    )(np0, np1, np2, mlp_b)


def _bound_body(s_ref, cnt_ref, b_out):
    s = s_ref[...]
    cnt = cnt_ref[0] + cnt_ref[1]
    amax = jnp.max(s)
    amin = jnp.min(s)
    w = (amax - amin) / 128.0 + 1e-30
    iot = lax.broadcasted_iota(jnp.int32, (1, 128), 1).astype(jnp.float32)
    nz = cnt > 0.0
    kmax = jnp.max(jnp.where(nz, iot, -1.0), axis=1, keepdims=True)
    kmin = jnp.min(jnp.where(nz, iot, 200.0), axis=1, keepdims=True)
    ub = amin + w * (kmax + 1.0)
    lb = amin + w * kmin
    b_out[...] = jnp.maximum(s * ub, s * lb)


def _bound(s, cparts):
    return pl.pallas_call(
        _bound_body,
        grid=(1,),
        in_specs=[
            pl.BlockSpec((N, 1), lambda i: (0, 0)),
            pl.BlockSpec((NC, N, DIM), lambda i: (0, 0, 0)),
        ],
        out_specs=pl.BlockSpec((N, 1), lambda i: (0, 0)),
        out_shape=jax.ShapeDtypeStruct((N, 1), jnp.float32),
    )(s, cparts)


def _final_body(s_ref, ep_ref, batch_ref, out):
    denom = ep_ref[0, :, 0:1] + ep_ref[1, :, 0:1]
    num = ep_ref[0, :, 1:2] + ep_ref[1, :, 1:2]
    new = jnp.where(denom > 0.0, num / denom, 0.0)
    v = s_ref[...] + new
    iot = lax.broadcasted_iota(jnp.int32, (1, NB), 1)
    oh = batch_ref[...] == iot
    neg = jnp.float32(-3.0e38)
    m_b = jnp.max(jnp.where(oh, v, neg), axis=0, keepdims=True)
    m_n = jnp.max(jnp.where(oh, m_b, neg), axis=1, keepdims=True)
    e = jnp.exp(v - m_n)
    d_b = jnp.sum(jnp.where(oh, e, 0.0), axis=0, keepdims=True)
    d_n = jnp.sum(jnp.where(oh, d_b, 0.0), axis=1, keepdims=True)
    out[...] = e / (d_n + 1e-16)


def _final(s, eparts, batch2d):
    return pl.pallas_call(
        _final_body,
        grid=(1,),
        in_specs=[
            pl.BlockSpec((N, 1), lambda i: (0, 0)),
            pl.BlockSpec((NC, N, 16), lambda i: (0, 0, 0)),
            pl.BlockSpec((N, 1), lambda i: (0, 0)),
        ],
        out_specs=pl.BlockSpec((N, 1), lambda i: (0, 0)),
        out_shape=jax.ShapeDtypeStruct((N, 1), jnp.float32),
    )(s, eparts, batch2d)


def kernel(x, edge_index, batch, W1_0, b1_0, W2_0, b2_0, W1_1, b1_1, W2_1,
           b2_1, W1_2, b1_2, W2_2, b2_2, mlp_W, mlp_b):
    src_a = edge_index[0].reshape(NW, NCA, CHA)
    dst_a = edge_index[1].reshape(NW, NCA, CHA)
    src_e = edge_index[0].reshape(NW, NCHUNK, CH)
    dst_e = edge_index[1].reshape(NW, NCHUNK, CH)
    layers = [(W1_0, b1_0, W2_0, b2_0), (W1_1, b1_1, W2_1, b2_1),
              (W1_2, b1_2, W2_2, b2_2)]

    h = x
    nps = []
    for i, (W1, b1, W2, b2) in enumerate(layers):
        parts = _sc_agg(h, src_a, dst_a)
        mw = mlp_W[0, i * DIM:(i + 1) * DIM].reshape(DIM, 1)
        h, np_i = _mlp(h, parts, W1.T, b1.reshape(1, DIM), W2.T,
                       b2.reshape(1, DIM), mw, last=(i == L - 1))
        nps.append(np_i)

    s, onehot = _prep(nps[0], nps[1], nps[2], mlp_b.reshape(1, 1))
    cparts = _sc_agg(onehot, src_a, dst_a)
    bound = _bound(s, cparts)
    eparts = _sc_edge(s.reshape(N), bound.reshape(N), src_e, dst_e)
    return _final(s, eparts, batch.reshape(N, 1))
